# baseline (speedup 1.0000x reference)
import jax
import jax.numpy as jnp
from jax.experimental import pallas as pl
from jax.experimental.pallas import tpu as pltpu

_BM = 625


def _gcn_body(t_ref, x_ref, f_ref, o_ref):
    tx = jnp.dot(t_ref[0], x_ref[...], preferred_element_type=jnp.float32)
    o_ref[0] = jnp.dot(tx, f_ref[...], preferred_element_type=jnp.float32)


def kernel(transform, x, filters):
    n, d = x.shape
    nf = filters.shape[1]
    g = n // _BM
    out = pl.pallas_call(
        _gcn_body,
        grid=(g,),
        in_specs=[
            pl.BlockSpec((1, _BM, n), lambda i: (i, 0, 0)),
            pl.BlockSpec((n, d), lambda i: (0, 0)),
            pl.BlockSpec((d, nf), lambda i: (0, 0)),
        ],
        out_specs=pl.BlockSpec((1, _BM, nf), lambda i: (i, 0, 0)),
        out_shape=jax.ShapeDtypeStruct((g, _BM, nf), jnp.float32),
        compiler_params=pltpu.CompilerParams(
            dimension_semantics=("parallel",),
        ),
    )(transform.reshape(g, _BM, n), x, filters)
    return out.reshape(n, nf)


# speedup vs baseline: 3.5880x; 3.5880x over previous
import jax
import jax.numpy as jnp
from jax.experimental import pallas as pl
from jax.experimental.pallas import tpu as pltpu

_BM = 400


def _gcn_body(t_ref, x_ref, f_ref, o_ref, xf_ref):
    @pl.when(pl.program_id(0) == 0)
    def _():
        xf_ref[...] = jnp.dot(x_ref[...], f_ref[...],
                              preferred_element_type=jnp.float32)

    o_ref[...] = jnp.dot(t_ref[...], xf_ref[...],
                         preferred_element_type=jnp.float32)


def kernel(transform, x, filters):
    n, d = x.shape
    nf = filters.shape[1]
    return pl.pallas_call(
        _gcn_body,
        grid=(n // _BM,),
        in_specs=[
            pl.BlockSpec((_BM, n), lambda i: (i, 0)),
            pl.BlockSpec((n, d), lambda i: (0, 0)),
            pl.BlockSpec((d, nf), lambda i: (0, 0)),
        ],
        out_specs=pl.BlockSpec((_BM, nf), lambda i: (i, 0)),
        out_shape=jax.ShapeDtypeStruct((n, nf), jnp.float32),
        scratch_shapes=[pltpu.VMEM((n, nf), jnp.float32)],
        compiler_params=pltpu.CompilerParams(
            dimension_semantics=("arbitrary",),
        ),
    )(transform, x, filters)
